# baseline (speedup 1.0000x reference)
import functools

import jax
import jax.numpy as jnp
from jax import lax
from jax.experimental import pallas as pl
from jax.experimental.pallas import tpu as pltpu
from jax.experimental.pallas import tpu_sc as plsc

NUM_CORES = 2
NUM_SUBCORES = 16
NUM_WORKERS = NUM_CORES * NUM_SUBCORES
CHUNK = 128
PACK = 4


@functools.partial(jax.jit, static_argnums=(2,))
def _sc_gather(table, idx3, b_per_w):
    n_chunks = idx3.shape[1]
    batch = NUM_WORKERS * b_per_w
    d = table.shape[1]
    mesh = plsc.VectorSubcoreMesh(core_axis_name="c", subcore_axis_name="s")

    @functools.partial(
        pl.kernel,
        mesh=mesh,
        out_type=jax.ShapeDtypeStruct((batch, d), jnp.float32),
        scratch_types=[
            pltpu.VMEM((n_chunks, CHUNK), jnp.int32),
            pltpu.VMEM((b_per_w, d), jnp.float32),
            pltpu.SemaphoreType.DMA,
        ],
    )
    def k(table_hbm, idx_hbm, out_hbm, idx_v, rows_v, sem):
        wid = lax.axis_index("s") * NUM_CORES + lax.axis_index("c")
        base = wid * b_per_w
        pltpu.sync_copy(idx_hbm.at[wid], idx_v)
        copies = []
        for j in range(n_chunks):
            copies.append(
                pltpu.async_copy(
                    table_hbm.at[idx_v.at[j]],
                    rows_v.at[pl.ds(j * CHUNK, CHUNK)],
                    sem,
                )
            )
        for c in copies:
            c.wait()
        pltpu.sync_copy(rows_v, out_hbm.at[pl.ds(base, b_per_w)])

    return k(table, idx3)


def _tc_merge(g128, rem2d, mode2d, ts, mode_table, w_user_stack, w_mode,
              w_time, time_W, time_b2, pref_b2):
    batch, d_g = g128.shape
    d_out = w_mode.shape[0]
    d_in = w_mode.shape[1]
    num_modes = mode_table.shape[0]
    blk = 2048
    grid = (batch // blk,)

    def body(g_ref, r_ref, m_ref, t_ref, mt_ref, ws_ref, wm_ref, wt_ref,
             tw_ref, tb_ref, pb_ref, o_ref):
        sel = (lax.broadcasted_iota(jnp.int32, (blk, d_g), 1) // d_in
               == r_ref[...]).astype(jnp.float32)
        user_c = lax.dot_general(g_ref[...] * sel, ws_ref[...],
                                 (((1,), (0,)), ((), ())),
                                 preferred_element_type=jnp.float32)
        oh = (lax.broadcasted_iota(jnp.int32, (blk, num_modes), 1)
              == m_ref[...]).astype(jnp.float32)
        m2 = lax.dot_general(mt_ref[...], wm_ref[...],
                             (((1,), (1,)), ((), ())),
                             preferred_element_type=jnp.float32)
        mode_c = lax.dot_general(oh, m2, (((1,), (0,)), ((), ())),
                                 preferred_element_type=jnp.float32)
        wc = lax.dot_general(wt_ref[...], tw_ref[...],
                             (((1,), (0,)), ((), ())),
                             preferred_element_type=jnp.float32)
        time_c = lax.dot_general(t_ref[...], wc, (((1,), (1,)), ((), ())),
                                 preferred_element_type=jnp.float32)
        bias = lax.dot_general(tb_ref[...], wt_ref[...],
                               (((1,), (1,)), ((), ())),
                               preferred_element_type=jnp.float32) + pb_ref[...]
        o_ref[...] = user_c + mode_c + time_c + bias

    return pl.pallas_call(
        body,
        grid=grid,
        in_specs=[
            pl.BlockSpec((blk, d_g), lambda i: (i, 0)),
            pl.BlockSpec((blk, 1), lambda i: (i, 0)),
            pl.BlockSpec((blk, 1), lambda i: (i, 0)),
            pl.BlockSpec((blk, ts.shape[1]), lambda i: (i, 0)),
            pl.BlockSpec((num_modes, d_in), lambda i: (0, 0)),
            pl.BlockSpec((d_g, d_out), lambda i: (0, 0)),
            pl.BlockSpec((d_out, d_in), lambda i: (0, 0)),
            pl.BlockSpec((d_out, d_in), lambda i: (0, 0)),
            pl.BlockSpec((d_in, ts.shape[1]), lambda i: (0, 0)),
            pl.BlockSpec((1, d_in), lambda i: (0, 0)),
            pl.BlockSpec((1, d_out), lambda i: (0, 0)),
        ],
        out_specs=pl.BlockSpec((blk, d_out), lambda i: (i, 0)),
        out_shape=jax.ShapeDtypeStruct((batch, d_out), jnp.float32),
    )(g128, rem2d, mode2d, ts, mode_table, w_user_stack, w_mode,
      w_time, time_W, time_b2, pref_b2)


def kernel(user_id, transport_mode, timestamp, user_table, mode_table,
           time_W, time_b, pref_W, pref_b):
    batch = user_id.shape[0]
    d = user_table.shape[1]
    d_out = pref_W.shape[0]
    b_per_w = batch // NUM_WORKERS
    table128 = user_table.reshape(user_table.shape[0] // PACK, PACK * d)
    idx3 = (user_id // PACK).reshape(NUM_WORKERS, b_per_w // CHUNK, CHUNK)
    rem2d = (user_id % PACK).reshape(batch, 1)
    g128 = _sc_gather(table128, idx3, b_per_w)
    w_user = pref_W[:, 0:d]
    w_user_stack = jnp.concatenate([w_user.T] * PACK, axis=0)
    return _tc_merge(
        g128,
        rem2d,
        transport_mode.reshape(batch, 1),
        timestamp,
        mode_table,
        w_user_stack,
        pref_W[:, d:2 * d],
        pref_W[:, 2 * d:3 * d],
        time_W,
        time_b.reshape(1, d),
        pref_b.reshape(1, d_out),
    )


# speedup vs baseline: 1.1229x; 1.1229x over previous
import functools

import jax
import jax.numpy as jnp
from jax import lax
from jax.experimental import pallas as pl
from jax.experimental.pallas import tpu as pltpu
from jax.experimental.pallas import tpu_sc as plsc

NUM_CORES = 2
NUM_SUBCORES = 16
NUM_WORKERS = NUM_CORES * NUM_SUBCORES
CHUNK = 128
PACK = 4


def _tc_repack(table_t):
    d, v = table_t.shape
    w = 4096
    grid = ((v + w - 1) // w,)

    def body(t_ref, o_ref):
        x = t_ref[...]
        eye = (lax.broadcasted_iota(jnp.int32, (d, d), 0)
               == lax.broadcasted_iota(jnp.int32, (d, d), 1)).astype(jnp.float32)
        xt = lax.dot_general(x, eye, (((0,), (0,)), ((), ())),
                             preferred_element_type=jnp.float32)
        x3 = xt.reshape(w // PACK, PACK, d)
        o_ref[...] = jnp.concatenate([x3[:, p, :] for p in range(PACK)], axis=1)

    return pl.pallas_call(
        body,
        grid=grid,
        in_specs=[pl.BlockSpec((d, w), lambda i: (0, i))],
        out_specs=pl.BlockSpec((w // PACK, PACK * d), lambda i: (i, 0)),
        out_shape=jax.ShapeDtypeStruct((grid[0] * (w // PACK), PACK * d),
                                       jnp.float32),
    )(table_t)


@functools.partial(jax.jit, static_argnums=(2,))
def _sc_gather(table, idx3, b_per_w):
    n_chunks = idx3.shape[1]
    batch = NUM_WORKERS * b_per_w
    d = table.shape[1]
    mesh = plsc.VectorSubcoreMesh(core_axis_name="c", subcore_axis_name="s")

    @functools.partial(
        pl.kernel,
        mesh=mesh,
        out_type=jax.ShapeDtypeStruct((batch, d), jnp.float32),
        scratch_types=[
            pltpu.VMEM((n_chunks, CHUNK), jnp.int32),
            pltpu.VMEM((b_per_w, d), jnp.float32),
            pltpu.SemaphoreType.DMA,
        ],
    )
    def k(table_hbm, idx_hbm, out_hbm, idx_v, rows_v, sem):
        wid = lax.axis_index("s") * NUM_CORES + lax.axis_index("c")
        base = wid * b_per_w
        pltpu.sync_copy(idx_hbm.at[wid], idx_v)
        copies = []
        for j in range(n_chunks):
            copies.append(
                pltpu.async_copy(
                    table_hbm.at[idx_v.at[j]],
                    rows_v.at[pl.ds(j * CHUNK, CHUNK)],
                    sem,
                )
            )
        for c in copies:
            c.wait()
        pltpu.sync_copy(rows_v, out_hbm.at[pl.ds(base, b_per_w)])

    return k(table, idx3)


def _tc_merge(g128, rem2d, mode2d, ts, mode_table, w_user_stack, w_mode,
              w_time, time_W, time_b2, pref_b2):
    batch, d_g = g128.shape
    d_out = w_mode.shape[0]
    d_in = w_mode.shape[1]
    num_modes = mode_table.shape[0]
    blk = 2048
    grid = (batch // blk,)

    def body(g_ref, r_ref, m_ref, t_ref, mt_ref, ws_ref, wm_ref, wt_ref,
             tw_ref, tb_ref, pb_ref, o_ref):
        sel = (lax.broadcasted_iota(jnp.int32, (blk, d_g), 1) // d_in
               == r_ref[...]).astype(jnp.float32)
        user_c = lax.dot_general(g_ref[...] * sel, ws_ref[...],
                                 (((1,), (0,)), ((), ())),
                                 preferred_element_type=jnp.float32)
        oh = (lax.broadcasted_iota(jnp.int32, (blk, num_modes), 1)
              == m_ref[...]).astype(jnp.float32)
        m2 = lax.dot_general(mt_ref[...], wm_ref[...],
                             (((1,), (1,)), ((), ())),
                             preferred_element_type=jnp.float32)
        mode_c = lax.dot_general(oh, m2, (((1,), (0,)), ((), ())),
                                 preferred_element_type=jnp.float32)
        wc = lax.dot_general(wt_ref[...], tw_ref[...],
                             (((1,), (0,)), ((), ())),
                             preferred_element_type=jnp.float32)
        time_c = lax.dot_general(t_ref[...], wc, (((1,), (1,)), ((), ())),
                                 preferred_element_type=jnp.float32)
        bias = lax.dot_general(tb_ref[...], wt_ref[...],
                               (((1,), (1,)), ((), ())),
                               preferred_element_type=jnp.float32) + pb_ref[...]
        o_ref[...] = user_c + mode_c + time_c + bias

    return pl.pallas_call(
        body,
        grid=grid,
        in_specs=[
            pl.BlockSpec((blk, d_g), lambda i: (i, 0)),
            pl.BlockSpec((blk, 1), lambda i: (i, 0)),
            pl.BlockSpec((blk, 1), lambda i: (i, 0)),
            pl.BlockSpec((blk, ts.shape[1]), lambda i: (i, 0)),
            pl.BlockSpec((num_modes, d_in), lambda i: (0, 0)),
            pl.BlockSpec((d_g, d_out), lambda i: (0, 0)),
            pl.BlockSpec((d_out, d_in), lambda i: (0, 0)),
            pl.BlockSpec((d_out, d_in), lambda i: (0, 0)),
            pl.BlockSpec((d_in, ts.shape[1]), lambda i: (0, 0)),
            pl.BlockSpec((1, d_in), lambda i: (0, 0)),
            pl.BlockSpec((1, d_out), lambda i: (0, 0)),
        ],
        out_specs=pl.BlockSpec((blk, d_out), lambda i: (i, 0)),
        out_shape=jax.ShapeDtypeStruct((batch, d_out), jnp.float32),
    )(g128, rem2d, mode2d, ts, mode_table, w_user_stack, w_mode,
      w_time, time_W, time_b2, pref_b2)


def kernel(user_id, transport_mode, timestamp, user_table, mode_table,
           time_W, time_b, pref_W, pref_b):
    batch = user_id.shape[0]
    d = user_table.shape[1]
    d_out = pref_W.shape[0]
    b_per_w = batch // NUM_WORKERS
    table128 = _tc_repack(user_table.T)
    idx3 = (user_id // PACK).reshape(NUM_WORKERS, b_per_w // CHUNK, CHUNK)
    rem2d = (user_id % PACK).reshape(batch, 1)
    g128 = _sc_gather(table128, idx3, b_per_w)
    w_user = pref_W[:, 0:d]
    w_user_stack = jnp.concatenate([w_user.T] * PACK, axis=0)
    return _tc_merge(
        g128,
        rem2d,
        transport_mode.reshape(batch, 1),
        timestamp,
        mode_table,
        w_user_stack,
        pref_W[:, d:2 * d],
        pref_W[:, 2 * d:3 * d],
        time_W,
        time_b.reshape(1, d),
        pref_b.reshape(1, d_out),
    )


# speedup vs baseline: 1.7501x; 1.5585x over previous
import functools

import jax
import jax.numpy as jnp
from jax import lax
from jax.experimental import pallas as pl
from jax.experimental.pallas import tpu as pltpu
from jax.experimental.pallas import tpu_sc as plsc

NUM_CORES = 2
NUM_SUBCORES = 16
NUM_WORKERS = NUM_CORES * NUM_SUBCORES
CHUNK = 128
PACK = 4


def _tc_repack(table_t):
    d, v = table_t.shape
    w = 4096
    grid = ((v + w - 1) // w,)

    c = w // PACK

    def body(t_ref, o_ref):
        ks = lax.broadcasted_iota(jnp.int32, (d, PACK * d), 0)
        cs = lax.broadcasted_iota(jnp.int32, (d, PACK * d), 1)
        acc = jnp.zeros((c, PACK * d), jnp.float32)
        for p in range(PACK):
            e_p = (cs == PACK * ks + p).astype(jnp.float32)
            acc = acc + lax.dot_general(
                t_ref[:, pl.ds(p * c, c)], e_p, (((0,), (0,)), ((), ())),
                preferred_element_type=jnp.float32)
        o_ref[...] = acc

    return pl.pallas_call(
        body,
        grid=grid,
        in_specs=[pl.BlockSpec((d, w), lambda i: (0, i))],
        out_specs=pl.BlockSpec((w // PACK, PACK * d), lambda i: (i, 0)),
        out_shape=jax.ShapeDtypeStruct((grid[0] * (w // PACK), PACK * d),
                                       jnp.float32),
    )(table_t)


@functools.partial(jax.jit, static_argnums=(2,))
def _sc_gather(table, idx3, b_per_w):
    n_chunks = idx3.shape[1]
    batch = NUM_WORKERS * b_per_w
    d = table.shape[1]
    mesh = plsc.VectorSubcoreMesh(core_axis_name="c", subcore_axis_name="s")

    @functools.partial(
        pl.kernel,
        mesh=mesh,
        out_type=jax.ShapeDtypeStruct((batch, d), jnp.float32),
        scratch_types=[
            pltpu.VMEM((n_chunks, CHUNK), jnp.int32),
            pltpu.VMEM((b_per_w, d), jnp.float32),
            pltpu.SemaphoreType.DMA,
        ],
    )
    def k(table_hbm, idx_hbm, out_hbm, idx_v, rows_v, sem):
        wid = lax.axis_index("s") * NUM_CORES + lax.axis_index("c")
        base = wid * b_per_w
        pltpu.sync_copy(idx_hbm.at[wid], idx_v)
        copies = []
        for j in range(n_chunks):
            copies.append(
                pltpu.async_copy(
                    table_hbm.at[idx_v.at[j]],
                    rows_v.at[pl.ds(j * CHUNK, CHUNK)],
                    sem,
                )
            )
        for c in copies:
            c.wait()
        pltpu.sync_copy(rows_v, out_hbm.at[pl.ds(base, b_per_w)])

    return k(table, idx3)


def _tc_merge(g128, rem2d, mode2d, ts, mode_table, w_user_stack, w_mode,
              w_time, time_W, time_b2, pref_b2):
    batch, d_g = g128.shape
    d_out = w_mode.shape[0]
    d_in = w_mode.shape[1]
    num_modes = mode_table.shape[0]
    blk = 2048
    grid = (batch // blk,)

    def body(g_ref, r_ref, m_ref, t_ref, mt_ref, ws_ref, wm_ref, wt_ref,
             tw_ref, tb_ref, pb_ref, o_ref):
        sel = (lax.broadcasted_iota(jnp.int32, (blk, d_g), 1) % PACK
               == r_ref[...]).astype(jnp.float32)
        user_c = lax.dot_general(g_ref[...] * sel, ws_ref[...],
                                 (((1,), (0,)), ((), ())),
                                 preferred_element_type=jnp.float32)
        oh = (lax.broadcasted_iota(jnp.int32, (blk, num_modes), 1)
              == m_ref[...]).astype(jnp.float32)
        m2 = lax.dot_general(mt_ref[...], wm_ref[...],
                             (((1,), (1,)), ((), ())),
                             preferred_element_type=jnp.float32)
        mode_c = lax.dot_general(oh, m2, (((1,), (0,)), ((), ())),
                                 preferred_element_type=jnp.float32)
        wc = lax.dot_general(wt_ref[...], tw_ref[...],
                             (((1,), (0,)), ((), ())),
                             preferred_element_type=jnp.float32)
        time_c = lax.dot_general(t_ref[...], wc, (((1,), (1,)), ((), ())),
                                 preferred_element_type=jnp.float32)
        bias = lax.dot_general(tb_ref[...], wt_ref[...],
                               (((1,), (1,)), ((), ())),
                               preferred_element_type=jnp.float32) + pb_ref[...]
        o_ref[...] = user_c + mode_c + time_c + bias

    return pl.pallas_call(
        body,
        grid=grid,
        in_specs=[
            pl.BlockSpec((blk, d_g), lambda i: (i, 0)),
            pl.BlockSpec((blk, 1), lambda i: (i, 0)),
            pl.BlockSpec((blk, 1), lambda i: (i, 0)),
            pl.BlockSpec((blk, ts.shape[1]), lambda i: (i, 0)),
            pl.BlockSpec((num_modes, d_in), lambda i: (0, 0)),
            pl.BlockSpec((d_g, d_out), lambda i: (0, 0)),
            pl.BlockSpec((d_out, d_in), lambda i: (0, 0)),
            pl.BlockSpec((d_out, d_in), lambda i: (0, 0)),
            pl.BlockSpec((d_in, ts.shape[1]), lambda i: (0, 0)),
            pl.BlockSpec((1, d_in), lambda i: (0, 0)),
            pl.BlockSpec((1, d_out), lambda i: (0, 0)),
        ],
        out_specs=pl.BlockSpec((blk, d_out), lambda i: (i, 0)),
        out_shape=jax.ShapeDtypeStruct((batch, d_out), jnp.float32),
    )(g128, rem2d, mode2d, ts, mode_table, w_user_stack, w_mode,
      w_time, time_W, time_b2, pref_b2)


def kernel(user_id, transport_mode, timestamp, user_table, mode_table,
           time_W, time_b, pref_W, pref_b):
    batch = user_id.shape[0]
    d = user_table.shape[1]
    d_out = pref_W.shape[0]
    b_per_w = batch // NUM_WORKERS
    table128 = _tc_repack(user_table.T)
    row_id = (user_id >> 12) * 1024 + (user_id & 1023)
    idx3 = row_id.reshape(NUM_WORKERS, b_per_w // CHUNK, CHUNK)
    rem2d = ((user_id >> 10) & 3).reshape(batch, 1)
    g128 = _sc_gather(table128, idx3, b_per_w)
    w_user = pref_W[:, 0:d]
    w_user_stack = jnp.repeat(w_user.T, PACK, axis=0)
    return _tc_merge(
        g128,
        rem2d,
        transport_mode.reshape(batch, 1),
        timestamp,
        mode_table,
        w_user_stack,
        pref_W[:, d:2 * d],
        pref_W[:, 2 * d:3 * d],
        time_W,
        time_b.reshape(1, d),
        pref_b.reshape(1, d_out),
    )


# speedup vs baseline: 2.0954x; 1.1973x over previous
import functools

import jax
import jax.numpy as jnp
from jax import lax
from jax.experimental import pallas as pl
from jax.experimental.pallas import tpu as pltpu
from jax.experimental.pallas import tpu_sc as plsc

NUM_CORES = 2
NUM_SUBCORES = 16
NUM_WORKERS = NUM_CORES * NUM_SUBCORES
CHUNK = 128
PACK = 4


def _tc_repack(table_t):
    d, v = table_t.shape
    w = 4096
    grid = ((v + w - 1) // w,)

    c = w // PACK

    def body(t_ref, o_ref):
        xs = jnp.concatenate(
            [t_ref[:, pl.ds(p * c, c)] for p in range(PACK)], axis=0
        )
        rows = lax.broadcasted_iota(jnp.int32, (PACK * d, PACK * d), 0)
        cols = lax.broadcasted_iota(jnp.int32, (PACK * d, PACK * d), 1)
        e = (cols == PACK * (rows % d) + rows // d).astype(jnp.float32)
        o_ref[...] = lax.dot_general(xs, e, (((0,), (0,)), ((), ())),
                                     preferred_element_type=jnp.float32)

    return pl.pallas_call(
        body,
        grid=grid,
        in_specs=[pl.BlockSpec((d, w), lambda i: (0, i))],
        out_specs=pl.BlockSpec((w // PACK, PACK * d), lambda i: (i, 0)),
        out_shape=jax.ShapeDtypeStruct((grid[0] * (w // PACK), PACK * d),
                                       jnp.float32),
    )(table_t)


@functools.partial(jax.jit, static_argnums=(2,))
def _sc_gather(table, idx3, b_per_w):
    n_chunks = idx3.shape[1]
    batch = NUM_WORKERS * b_per_w
    d = table.shape[1]
    mesh = plsc.VectorSubcoreMesh(core_axis_name="c", subcore_axis_name="s")

    @functools.partial(
        pl.kernel,
        mesh=mesh,
        out_type=jax.ShapeDtypeStruct((batch, d), jnp.float32),
        scratch_types=[
            pltpu.VMEM((n_chunks, CHUNK), jnp.int32),
            pltpu.VMEM((b_per_w, d), jnp.float32),
            pltpu.SemaphoreType.DMA,
        ],
    )
    def k(table_hbm, idx_hbm, out_hbm, idx_v, rows_v, sem):
        wid = lax.axis_index("s") * NUM_CORES + lax.axis_index("c")
        base = wid * b_per_w
        pltpu.sync_copy(idx_hbm.at[wid], idx_v)
        copies = []
        for j in range(n_chunks):
            copies.append(
                pltpu.async_copy(
                    table_hbm.at[idx_v.at[j]],
                    rows_v.at[pl.ds(j * CHUNK, CHUNK)],
                    sem,
                )
            )
        for c in copies:
            c.wait()
        pltpu.sync_copy(rows_v, out_hbm.at[pl.ds(base, b_per_w)])

    return k(table, idx3)


def _tc_merge(g128, rem2d, mode2d, ts, mode_table, w_user_stack, w_mode,
              w_time, time_W, time_b2, pref_b2):
    batch, d_g = g128.shape
    d_out = w_mode.shape[0]
    d_in = w_mode.shape[1]
    num_modes = mode_table.shape[0]
    blk = 2048
    grid = (batch // blk,)

    def body(g_ref, r_ref, m_ref, t_ref, mt_ref, ws_ref, wm_ref, wt_ref,
             tw_ref, tb_ref, pb_ref, o_ref):
        sel = (lax.broadcasted_iota(jnp.int32, (blk, d_g), 1) % PACK
               == r_ref[...]).astype(jnp.float32)
        user_c = lax.dot_general(g_ref[...] * sel, ws_ref[...],
                                 (((1,), (0,)), ((), ())),
                                 preferred_element_type=jnp.float32)
        oh = (lax.broadcasted_iota(jnp.int32, (blk, num_modes), 1)
              == m_ref[...]).astype(jnp.float32)
        m2 = lax.dot_general(mt_ref[...], wm_ref[...],
                             (((1,), (1,)), ((), ())),
                             preferred_element_type=jnp.float32)
        mode_c = lax.dot_general(oh, m2, (((1,), (0,)), ((), ())),
                                 preferred_element_type=jnp.float32)
        wc = lax.dot_general(wt_ref[...], tw_ref[...],
                             (((1,), (0,)), ((), ())),
                             preferred_element_type=jnp.float32)
        time_c = lax.dot_general(t_ref[...], wc, (((1,), (1,)), ((), ())),
                                 preferred_element_type=jnp.float32)
        bias = lax.dot_general(tb_ref[...], wt_ref[...],
                               (((1,), (1,)), ((), ())),
                               preferred_element_type=jnp.float32) + pb_ref[...]
        o_ref[...] = user_c + mode_c + time_c + bias

    return pl.pallas_call(
        body,
        grid=grid,
        in_specs=[
            pl.BlockSpec((blk, d_g), lambda i: (i, 0)),
            pl.BlockSpec((blk, 1), lambda i: (i, 0)),
            pl.BlockSpec((blk, 1), lambda i: (i, 0)),
            pl.BlockSpec((blk, ts.shape[1]), lambda i: (i, 0)),
            pl.BlockSpec((num_modes, d_in), lambda i: (0, 0)),
            pl.BlockSpec((d_g, d_out), lambda i: (0, 0)),
            pl.BlockSpec((d_out, d_in), lambda i: (0, 0)),
            pl.BlockSpec((d_out, d_in), lambda i: (0, 0)),
            pl.BlockSpec((d_in, ts.shape[1]), lambda i: (0, 0)),
            pl.BlockSpec((1, d_in), lambda i: (0, 0)),
            pl.BlockSpec((1, d_out), lambda i: (0, 0)),
        ],
        out_specs=pl.BlockSpec((blk, d_out), lambda i: (i, 0)),
        out_shape=jax.ShapeDtypeStruct((batch, d_out), jnp.float32),
    )(g128, rem2d, mode2d, ts, mode_table, w_user_stack, w_mode,
      w_time, time_W, time_b2, pref_b2)


def kernel(user_id, transport_mode, timestamp, user_table, mode_table,
           time_W, time_b, pref_W, pref_b):
    batch = user_id.shape[0]
    d = user_table.shape[1]
    d_out = pref_W.shape[0]
    b_per_w = batch // NUM_WORKERS
    table128 = _tc_repack(user_table.T)
    row_id = (user_id >> 12) * 1024 + (user_id & 1023)
    idx3 = row_id.reshape(NUM_WORKERS, b_per_w // CHUNK, CHUNK)
    rem2d = ((user_id >> 10) & 3).reshape(batch, 1)
    g128 = _sc_gather(table128, idx3, b_per_w)
    w_user = pref_W[:, 0:d]
    w_user_stack = jnp.repeat(w_user.T, PACK, axis=0)
    return _tc_merge(
        g128,
        rem2d,
        transport_mode.reshape(batch, 1),
        timestamp,
        mode_table,
        w_user_stack,
        pref_W[:, d:2 * d],
        pref_W[:, 2 * d:3 * d],
        time_W,
        time_b.reshape(1, d),
        pref_b.reshape(1, d_out),
    )


# speedup vs baseline: 3.4890x; 1.6651x over previous
import functools

import jax
import jax.numpy as jnp
from jax import lax
from jax.experimental import pallas as pl
from jax.experimental.pallas import tpu as pltpu
from jax.experimental.pallas import tpu_sc as plsc

NUM_CORES = 2
NUM_SUBCORES = 16
NUM_WORKERS = NUM_CORES * NUM_SUBCORES
CHUNK = 128
PACK = 4


def _tc_repack(table_t):
    d, v = table_t.shape
    w = 16384
    grid = ((v + w - 1) // w,)

    c = w // PACK

    def body(t_ref, o_ref):
        xs = jnp.concatenate(
            [t_ref[:, pl.ds(p * c, c)] for p in range(PACK)], axis=0
        )
        rows = lax.broadcasted_iota(jnp.int32, (PACK * d, PACK * d), 0)
        cols = lax.broadcasted_iota(jnp.int32, (PACK * d, PACK * d), 1)
        e = (cols == PACK * (rows % d) + rows // d).astype(jnp.float32)
        o_ref[...] = lax.dot_general(xs, e, (((0,), (0,)), ((), ())),
                                     preferred_element_type=jnp.float32)

    return pl.pallas_call(
        body,
        grid=grid,
        in_specs=[pl.BlockSpec((d, w), lambda i: (0, i))],
        out_specs=pl.BlockSpec((w // PACK, PACK * d), lambda i: (i, 0)),
        out_shape=jax.ShapeDtypeStruct((grid[0] * (w // PACK), PACK * d),
                                       jnp.float32),
    )(table_t)


@functools.partial(jax.jit, static_argnums=(2,))
def _sc_gather(table, idx3, b_per_w):
    n_chunks = idx3.shape[1]
    batch = NUM_WORKERS * b_per_w
    d = table.shape[1]
    mesh = plsc.VectorSubcoreMesh(core_axis_name="c", subcore_axis_name="s")

    @functools.partial(
        pl.kernel,
        mesh=mesh,
        out_type=jax.ShapeDtypeStruct((batch, d), jnp.float32),
        scratch_types=[
            pltpu.VMEM((n_chunks, CHUNK), jnp.int32),
            pltpu.VMEM((b_per_w, d), jnp.float32),
            pltpu.SemaphoreType.DMA,
        ],
    )
    def k(table_hbm, idx_hbm, out_hbm, idx_v, rows_v, sem):
        wid = lax.axis_index("s") * NUM_CORES + lax.axis_index("c")
        base = wid * b_per_w
        pltpu.sync_copy(idx_hbm.at[wid], idx_v)
        copies = []
        for j in range(n_chunks):
            copies.append(
                pltpu.async_copy(
                    table_hbm.at[idx_v.at[j]],
                    rows_v.at[pl.ds(j * CHUNK, CHUNK)],
                    sem,
                )
            )
        for c in copies:
            c.wait()
        pltpu.sync_copy(rows_v, out_hbm.at[pl.ds(base, b_per_w)])

    return k(table, idx3)


def _tc_merge(g128, rem2d, mode2d, ts, mode_table, w_user_stack, w_mode,
              w_time, time_W, time_b2, pref_b2):
    batch, d_g = g128.shape
    d_out = w_mode.shape[0]
    d_in = w_mode.shape[1]
    num_modes = mode_table.shape[0]
    blk = 2048
    grid = (batch // blk,)

    def body(g_ref, r_ref, m_ref, t_ref, mt_ref, ws_ref, wm_ref, wt_ref,
             tw_ref, tb_ref, pb_ref, o_ref):
        sel = (lax.broadcasted_iota(jnp.int32, (blk, d_g), 1) % PACK
               == r_ref[...]).astype(jnp.float32)
        user_c = lax.dot_general(g_ref[...] * sel, ws_ref[...],
                                 (((1,), (0,)), ((), ())),
                                 preferred_element_type=jnp.float32)
        oh = (lax.broadcasted_iota(jnp.int32, (blk, num_modes), 1)
              == m_ref[...]).astype(jnp.float32)
        m2 = lax.dot_general(mt_ref[...], wm_ref[...],
                             (((1,), (1,)), ((), ())),
                             preferred_element_type=jnp.float32)
        mode_c = lax.dot_general(oh, m2, (((1,), (0,)), ((), ())),
                                 preferred_element_type=jnp.float32)
        wc = lax.dot_general(wt_ref[...], tw_ref[...],
                             (((1,), (0,)), ((), ())),
                             preferred_element_type=jnp.float32)
        time_c = lax.dot_general(t_ref[...], wc, (((1,), (1,)), ((), ())),
                                 preferred_element_type=jnp.float32)
        bias = lax.dot_general(tb_ref[...], wt_ref[...],
                               (((1,), (1,)), ((), ())),
                               preferred_element_type=jnp.float32) + pb_ref[...]
        o_ref[...] = user_c + mode_c + time_c + bias

    return pl.pallas_call(
        body,
        grid=grid,
        in_specs=[
            pl.BlockSpec((blk, d_g), lambda i: (i, 0)),
            pl.BlockSpec((blk, 1), lambda i: (i, 0)),
            pl.BlockSpec((blk, 1), lambda i: (i, 0)),
            pl.BlockSpec((blk, ts.shape[1]), lambda i: (i, 0)),
            pl.BlockSpec((num_modes, d_in), lambda i: (0, 0)),
            pl.BlockSpec((d_g, d_out), lambda i: (0, 0)),
            pl.BlockSpec((d_out, d_in), lambda i: (0, 0)),
            pl.BlockSpec((d_out, d_in), lambda i: (0, 0)),
            pl.BlockSpec((d_in, ts.shape[1]), lambda i: (0, 0)),
            pl.BlockSpec((1, d_in), lambda i: (0, 0)),
            pl.BlockSpec((1, d_out), lambda i: (0, 0)),
        ],
        out_specs=pl.BlockSpec((blk, d_out), lambda i: (i, 0)),
        out_shape=jax.ShapeDtypeStruct((batch, d_out), jnp.float32),
    )(g128, rem2d, mode2d, ts, mode_table, w_user_stack, w_mode,
      w_time, time_W, time_b2, pref_b2)


def kernel(user_id, transport_mode, timestamp, user_table, mode_table,
           time_W, time_b, pref_W, pref_b):
    batch = user_id.shape[0]
    d = user_table.shape[1]
    d_out = pref_W.shape[0]
    b_per_w = batch // NUM_WORKERS
    table128 = _tc_repack(user_table.T)
    row_id = (user_id >> 14) * 4096 + (user_id & 4095)
    idx3 = row_id.reshape(NUM_WORKERS, b_per_w // CHUNK, CHUNK)
    rem2d = ((user_id >> 12) & 3).reshape(batch, 1)
    g128 = _sc_gather(table128, idx3, b_per_w)
    w_user = pref_W[:, 0:d]
    w_user_stack = jnp.repeat(w_user.T, PACK, axis=0)
    return _tc_merge(
        g128,
        rem2d,
        transport_mode.reshape(batch, 1),
        timestamp,
        mode_table,
        w_user_stack,
        pref_W[:, d:2 * d],
        pref_W[:, 2 * d:3 * d],
        time_W,
        time_b.reshape(1, d),
        pref_b.reshape(1, d_out),
    )


# speedup vs baseline: 3.8473x; 1.1027x over previous
import functools

import jax
import jax.numpy as jnp
from jax import lax
from jax.experimental import pallas as pl
from jax.experimental.pallas import tpu as pltpu
from jax.experimental.pallas import tpu_sc as plsc

NUM_CORES = 2
NUM_SUBCORES = 16
NUM_WORKERS = NUM_CORES * NUM_SUBCORES
CHUNK = 128
PACK = 4


def _tc_repack(table_t):
    d, v = table_t.shape
    w = 32768
    grid = ((v + w - 1) // w,)

    c = w // PACK

    def body(t_ref, o_ref):
        xs = jnp.concatenate(
            [t_ref[:, pl.ds(p * c, c)] for p in range(PACK)], axis=0
        )
        rows = lax.broadcasted_iota(jnp.int32, (PACK * d, PACK * d), 0)
        cols = lax.broadcasted_iota(jnp.int32, (PACK * d, PACK * d), 1)
        e = (cols == PACK * (rows % d) + rows // d).astype(jnp.float32)
        o_ref[...] = lax.dot_general(xs, e, (((0,), (0,)), ((), ())),
                                     preferred_element_type=jnp.float32)

    return pl.pallas_call(
        body,
        grid=grid,
        in_specs=[pl.BlockSpec((d, w), lambda i: (0, i))],
        out_specs=pl.BlockSpec((w // PACK, PACK * d), lambda i: (i, 0)),
        out_shape=jax.ShapeDtypeStruct((grid[0] * (w // PACK), PACK * d),
                                       jnp.float32),
    )(table_t)


@functools.partial(jax.jit, static_argnums=(2,))
def _sc_gather(table, idx3, b_per_w):
    n_chunks = idx3.shape[1]
    batch = NUM_WORKERS * b_per_w
    d = table.shape[1]
    mesh = plsc.VectorSubcoreMesh(core_axis_name="c", subcore_axis_name="s")

    @functools.partial(
        pl.kernel,
        mesh=mesh,
        out_type=jax.ShapeDtypeStruct((batch, d), jnp.float32),
        scratch_types=[
            pltpu.VMEM((n_chunks, CHUNK), jnp.int32),
            pltpu.VMEM((b_per_w, d), jnp.float32),
            pltpu.SemaphoreType.DMA,
        ],
    )
    def k(table_hbm, idx_hbm, out_hbm, idx_v, rows_v, sem):
        wid = lax.axis_index("s") * NUM_CORES + lax.axis_index("c")
        base = wid * b_per_w
        pltpu.sync_copy(idx_hbm.at[wid], idx_v)
        copies = []
        for j in range(n_chunks):
            copies.append(
                pltpu.async_copy(
                    table_hbm.at[idx_v.at[j]],
                    rows_v.at[pl.ds(j * CHUNK, CHUNK)],
                    sem,
                )
            )
        for c in copies:
            c.wait()
        pltpu.sync_copy(rows_v, out_hbm.at[pl.ds(base, b_per_w)])

    return k(table, idx3)


def _tc_merge(g128, rem2d, mode2d, ts, mode_table, w_user_stack, w_mode,
              w_time, time_W, time_b2, pref_b2):
    batch, d_g = g128.shape
    d_out = w_mode.shape[0]
    d_in = w_mode.shape[1]
    num_modes = mode_table.shape[0]
    blk = 2048
    grid = (batch // blk,)

    def body(g_ref, r_ref, m_ref, t_ref, mt_ref, ws_ref, wm_ref, wt_ref,
             tw_ref, tb_ref, pb_ref, o_ref):
        sel = (lax.broadcasted_iota(jnp.int32, (blk, d_g), 1) % PACK
               == r_ref[...]).astype(jnp.float32)
        user_c = lax.dot_general(g_ref[...] * sel, ws_ref[...],
                                 (((1,), (0,)), ((), ())),
                                 preferred_element_type=jnp.float32)
        oh = (lax.broadcasted_iota(jnp.int32, (blk, num_modes), 1)
              == m_ref[...]).astype(jnp.float32)
        m2 = lax.dot_general(mt_ref[...], wm_ref[...],
                             (((1,), (1,)), ((), ())),
                             preferred_element_type=jnp.float32)
        mode_c = lax.dot_general(oh, m2, (((1,), (0,)), ((), ())),
                                 preferred_element_type=jnp.float32)
        wc = lax.dot_general(wt_ref[...], tw_ref[...],
                             (((1,), (0,)), ((), ())),
                             preferred_element_type=jnp.float32)
        time_c = lax.dot_general(t_ref[...], wc, (((1,), (1,)), ((), ())),
                                 preferred_element_type=jnp.float32)
        bias = lax.dot_general(tb_ref[...], wt_ref[...],
                               (((1,), (1,)), ((), ())),
                               preferred_element_type=jnp.float32) + pb_ref[...]
        o_ref[...] = user_c + mode_c + time_c + bias

    return pl.pallas_call(
        body,
        grid=grid,
        in_specs=[
            pl.BlockSpec((blk, d_g), lambda i: (i, 0)),
            pl.BlockSpec((blk, 1), lambda i: (i, 0)),
            pl.BlockSpec((blk, 1), lambda i: (i, 0)),
            pl.BlockSpec((blk, ts.shape[1]), lambda i: (i, 0)),
            pl.BlockSpec((num_modes, d_in), lambda i: (0, 0)),
            pl.BlockSpec((d_g, d_out), lambda i: (0, 0)),
            pl.BlockSpec((d_out, d_in), lambda i: (0, 0)),
            pl.BlockSpec((d_out, d_in), lambda i: (0, 0)),
            pl.BlockSpec((d_in, ts.shape[1]), lambda i: (0, 0)),
            pl.BlockSpec((1, d_in), lambda i: (0, 0)),
            pl.BlockSpec((1, d_out), lambda i: (0, 0)),
        ],
        out_specs=pl.BlockSpec((blk, d_out), lambda i: (i, 0)),
        out_shape=jax.ShapeDtypeStruct((batch, d_out), jnp.float32),
    )(g128, rem2d, mode2d, ts, mode_table, w_user_stack, w_mode,
      w_time, time_W, time_b2, pref_b2)


def kernel(user_id, transport_mode, timestamp, user_table, mode_table,
           time_W, time_b, pref_W, pref_b):
    batch = user_id.shape[0]
    d = user_table.shape[1]
    d_out = pref_W.shape[0]
    b_per_w = batch // NUM_WORKERS
    table128 = _tc_repack(user_table.T)
    row_id = (user_id >> 15) * 8192 + (user_id & 8191)
    idx3 = row_id.reshape(NUM_WORKERS, b_per_w // CHUNK, CHUNK)
    rem2d = ((user_id >> 13) & 3).reshape(batch, 1)
    g128 = _sc_gather(table128, idx3, b_per_w)
    w_user = pref_W[:, 0:d]
    w_user_stack = jnp.repeat(w_user.T, PACK, axis=0)
    return _tc_merge(
        g128,
        rem2d,
        transport_mode.reshape(batch, 1),
        timestamp,
        mode_table,
        w_user_stack,
        pref_W[:, d:2 * d],
        pref_W[:, 2 * d:3 * d],
        time_W,
        time_b.reshape(1, d),
        pref_b.reshape(1, d_out),
    )


# speedup vs baseline: 3.8634x; 1.0042x over previous
import functools

import jax
import jax.numpy as jnp
from jax import lax
from jax.experimental import pallas as pl
from jax.experimental.pallas import tpu as pltpu
from jax.experimental.pallas import tpu_sc as plsc

NUM_CORES = 2
NUM_SUBCORES = 16
NUM_WORKERS = NUM_CORES * NUM_SUBCORES
CHUNK = 128
PACK = 4


def _tc_repack(table_t):
    d, v = table_t.shape
    w = 65536
    grid = ((v + w - 1) // w,)

    c = w // PACK

    def body(t_ref, o_ref):
        xs = jnp.concatenate(
            [t_ref[:, pl.ds(p * c, c)] for p in range(PACK)], axis=0
        )
        rows = lax.broadcasted_iota(jnp.int32, (PACK * d, PACK * d), 0)
        cols = lax.broadcasted_iota(jnp.int32, (PACK * d, PACK * d), 1)
        e = (cols == PACK * (rows % d) + rows // d).astype(jnp.float32)
        o_ref[...] = lax.dot_general(xs, e, (((0,), (0,)), ((), ())),
                                     preferred_element_type=jnp.float32)

    return pl.pallas_call(
        body,
        grid=grid,
        in_specs=[pl.BlockSpec((d, w), lambda i: (0, i))],
        out_specs=pl.BlockSpec((w // PACK, PACK * d), lambda i: (i, 0)),
        out_shape=jax.ShapeDtypeStruct((grid[0] * (w // PACK), PACK * d),
                                       jnp.float32),
    )(table_t)


@functools.partial(jax.jit, static_argnums=(2,))
def _sc_gather(table, idx3, b_per_w):
    n_chunks = idx3.shape[1]
    batch = NUM_WORKERS * b_per_w
    d = table.shape[1]
    mesh = plsc.VectorSubcoreMesh(core_axis_name="c", subcore_axis_name="s")

    @functools.partial(
        pl.kernel,
        mesh=mesh,
        out_type=jax.ShapeDtypeStruct((batch, d), jnp.float32),
        scratch_types=[
            pltpu.VMEM((n_chunks, CHUNK), jnp.int32),
            pltpu.VMEM((b_per_w, d), jnp.float32),
            pltpu.SemaphoreType.DMA,
        ],
    )
    def k(table_hbm, idx_hbm, out_hbm, idx_v, rows_v, sem):
        wid = lax.axis_index("s") * NUM_CORES + lax.axis_index("c")
        base = wid * b_per_w
        pltpu.sync_copy(idx_hbm.at[wid], idx_v)
        copies = []
        for j in range(n_chunks):
            copies.append(
                pltpu.async_copy(
                    table_hbm.at[idx_v.at[j]],
                    rows_v.at[pl.ds(j * CHUNK, CHUNK)],
                    sem,
                )
            )
        for c in copies:
            c.wait()
        pltpu.sync_copy(rows_v, out_hbm.at[pl.ds(base, b_per_w)])

    return k(table, idx3)


def _tc_merge(g128, rem2d, mode2d, ts, mode_table, w_user_stack, w_mode,
              w_time, time_W, time_b2, pref_b2):
    batch, d_g = g128.shape
    d_out = w_mode.shape[0]
    d_in = w_mode.shape[1]
    num_modes = mode_table.shape[0]
    blk = 2048
    grid = (batch // blk,)

    def body(g_ref, r_ref, m_ref, t_ref, mt_ref, ws_ref, wm_ref, wt_ref,
             tw_ref, tb_ref, pb_ref, o_ref):
        sel = (lax.broadcasted_iota(jnp.int32, (blk, d_g), 1) % PACK
               == r_ref[...]).astype(jnp.float32)
        user_c = lax.dot_general(g_ref[...] * sel, ws_ref[...],
                                 (((1,), (0,)), ((), ())),
                                 preferred_element_type=jnp.float32)
        oh = (lax.broadcasted_iota(jnp.int32, (blk, num_modes), 1)
              == m_ref[...]).astype(jnp.float32)
        m2 = lax.dot_general(mt_ref[...], wm_ref[...],
                             (((1,), (1,)), ((), ())),
                             preferred_element_type=jnp.float32)
        mode_c = lax.dot_general(oh, m2, (((1,), (0,)), ((), ())),
                                 preferred_element_type=jnp.float32)
        wc = lax.dot_general(wt_ref[...], tw_ref[...],
                             (((1,), (0,)), ((), ())),
                             preferred_element_type=jnp.float32)
        time_c = lax.dot_general(t_ref[...], wc, (((1,), (1,)), ((), ())),
                                 preferred_element_type=jnp.float32)
        bias = lax.dot_general(tb_ref[...], wt_ref[...],
                               (((1,), (1,)), ((), ())),
                               preferred_element_type=jnp.float32) + pb_ref[...]
        o_ref[...] = user_c + mode_c + time_c + bias

    return pl.pallas_call(
        body,
        grid=grid,
        in_specs=[
            pl.BlockSpec((blk, d_g), lambda i: (i, 0)),
            pl.BlockSpec((blk, 1), lambda i: (i, 0)),
            pl.BlockSpec((blk, 1), lambda i: (i, 0)),
            pl.BlockSpec((blk, ts.shape[1]), lambda i: (i, 0)),
            pl.BlockSpec((num_modes, d_in), lambda i: (0, 0)),
            pl.BlockSpec((d_g, d_out), lambda i: (0, 0)),
            pl.BlockSpec((d_out, d_in), lambda i: (0, 0)),
            pl.BlockSpec((d_out, d_in), lambda i: (0, 0)),
            pl.BlockSpec((d_in, ts.shape[1]), lambda i: (0, 0)),
            pl.BlockSpec((1, d_in), lambda i: (0, 0)),
            pl.BlockSpec((1, d_out), lambda i: (0, 0)),
        ],
        out_specs=pl.BlockSpec((blk, d_out), lambda i: (i, 0)),
        out_shape=jax.ShapeDtypeStruct((batch, d_out), jnp.float32),
    )(g128, rem2d, mode2d, ts, mode_table, w_user_stack, w_mode,
      w_time, time_W, time_b2, pref_b2)


def kernel(user_id, transport_mode, timestamp, user_table, mode_table,
           time_W, time_b, pref_W, pref_b):
    batch = user_id.shape[0]
    d = user_table.shape[1]
    d_out = pref_W.shape[0]
    b_per_w = batch // NUM_WORKERS
    table128 = _tc_repack(user_table.T)
    row_id = (user_id >> 16) * 16384 + (user_id & 16383)
    idx3 = row_id.reshape(NUM_WORKERS, b_per_w // CHUNK, CHUNK)
    rem2d = ((user_id >> 14) & 3).reshape(batch, 1)
    g128 = _sc_gather(table128, idx3, b_per_w)
    w_user = pref_W[:, 0:d]
    w_user_stack = jnp.repeat(w_user.T, PACK, axis=0)
    return _tc_merge(
        g128,
        rem2d,
        transport_mode.reshape(batch, 1),
        timestamp,
        mode_table,
        w_user_stack,
        pref_W[:, d:2 * d],
        pref_W[:, 2 * d:3 * d],
        time_W,
        time_b.reshape(1, d),
        pref_b.reshape(1, d_out),
    )


# speedup vs baseline: 4.5647x; 1.1815x over previous
import functools

import jax
import jax.numpy as jnp
from jax import lax
from jax.experimental import pallas as pl
from jax.experimental.pallas import tpu as pltpu
from jax.experimental.pallas import tpu_sc as plsc

NUM_CORES = 2
NUM_SUBCORES = 16
NUM_WORKERS = NUM_CORES * NUM_SUBCORES
CHUNK = 128
PACK = 4
W_BLK = 65536
C_BLK = W_BLK // PACK
L = 16


def _tc_repack(table_t):
    d, v = table_t.shape
    w = W_BLK
    grid = ((v + w - 1) // w,)
    c = w // PACK

    def body(t_ref, o_ref):
        xs = jnp.concatenate(
            [t_ref[:, pl.ds(p * c, c)] for p in range(PACK)], axis=0
        )
        rows = lax.broadcasted_iota(jnp.int32, (PACK * d, PACK * d), 0)
        cols = lax.broadcasted_iota(jnp.int32, (PACK * d, PACK * d), 1)
        e = (cols == PACK * (rows % d) + rows // d).astype(jnp.float32)
        o_ref[...] = lax.dot_general(xs, e, (((0,), (0,)), ((), ())),
                                     preferred_element_type=jnp.float32)

    return pl.pallas_call(
        body,
        grid=grid,
        in_specs=[pl.BlockSpec((d, w), lambda i: (0, i))],
        out_specs=pl.BlockSpec((w // PACK, PACK * d), lambda i: (i, 0)),
        out_shape=jax.ShapeDtypeStruct((grid[0] * (w // PACK), PACK * d),
                                       jnp.float32),
    )(table_t)


@jax.jit
def _sc_gather_compact(table128, uid3):
    n_chunks = uid3.shape[1]
    b_per_w = n_chunks * CHUNK
    batch = NUM_WORKERS * b_per_w
    d = 32
    mesh = plsc.VectorSubcoreMesh(core_axis_name="c", subcore_axis_name="s")

    @functools.partial(
        pl.kernel,
        mesh=mesh,
        compiler_params=pltpu.CompilerParams(needs_layout_passes=False),
        out_type=jax.ShapeDtypeStruct((d, batch), jnp.float32),
        scratch_types=[
            pltpu.VMEM((n_chunks, CHUNK), jnp.int32),
            pltpu.VMEM((n_chunks, CHUNK), jnp.int32),
            pltpu.VMEM((b_per_w, PACK * d), jnp.float32),
            pltpu.VMEM((d, b_per_w), jnp.float32),
            pltpu.SemaphoreType.DMA,
        ],
    )
    def k(table_hbm, uid_hbm, out_hbm, uid_v, rid_v, rows_v, outt_v, sem):
        wid = lax.axis_index("s") * NUM_CORES + lax.axis_index("c")
        base = wid * b_per_w
        pltpu.sync_copy(uid_hbm.at[wid], uid_v)
        for c in range(n_chunks):
            for i in range(CHUNK // L):
                u = uid_v[c, pl.ds(i * L, L)]
                rid_v[c, pl.ds(i * L, L)] = (
                    (u >> 16) * C_BLK + (u & (C_BLK - 1)))
        copies = []
        for c in range(n_chunks):
            copies.append(
                pltpu.async_copy(
                    table_hbm.at[rid_v.at[c]],
                    rows_v.at[pl.ds(c * CHUNK, CHUNK)],
                    sem,
                )
            )
        for cp in copies:
            cp.wait()
        kiota = lax.broadcasted_iota(jnp.int32, (L,), 0)
        for c in range(n_chunks):
            for i in range(CHUNK // L):
                rowbase = c * CHUNK + i * L
                u = uid_v[c, pl.ds(i * L, L)]
                p = (u >> 14) & 3
                rows16 = kiota + rowbase
                for kk in range(d):
                    vals = plsc.load_gather(rows_v, [rows16, PACK * kk + p])
                    outt_v[kk, pl.ds(rowbase, L)] = vals
        pltpu.sync_copy(outt_v, out_hbm.at[:, pl.ds(base, b_per_w)])

    return k(table128, uid3)


def _tc_merge_t(u_t, mode_t, ts_t, mode_table, w_user, w_mode, w_time,
                time_W, time_b_c, pref_b_c):
    d_in, batch = u_t.shape
    d_out = w_user.shape[0]
    num_modes = mode_table.shape[0]
    blk = 4096
    grid = (batch // blk,)

    def body(u_ref, m_ref, t_ref, mt_ref, wu_ref, wm_ref, wt_ref,
             tw_ref, tb_ref, pb_ref, o_ref):
        user_c = lax.dot_general(wu_ref[...], u_ref[...],
                                 (((1,), (0,)), ((), ())),
                                 preferred_element_type=jnp.float32)
        oh = (lax.broadcasted_iota(jnp.int32, (num_modes, blk), 0)
              .astype(jnp.float32) == m_ref[...]).astype(jnp.float32)
        m2 = lax.dot_general(wm_ref[...], mt_ref[...],
                             (((1,), (1,)), ((), ())),
                             preferred_element_type=jnp.float32)
        mode_c = lax.dot_general(m2, oh, (((1,), (0,)), ((), ())),
                                 preferred_element_type=jnp.float32)
        wc = lax.dot_general(wt_ref[...], tw_ref[...],
                             (((1,), (0,)), ((), ())),
                             preferred_element_type=jnp.float32)
        time_c = lax.dot_general(wc, t_ref[...], (((1,), (0,)), ((), ())),
                                 preferred_element_type=jnp.float32)
        bias = lax.dot_general(wt_ref[...], tb_ref[...],
                               (((1,), (0,)), ((), ())),
                               preferred_element_type=jnp.float32) + pb_ref[...]
        o_ref[...] = user_c + mode_c + time_c + bias

    return pl.pallas_call(
        body,
        grid=grid,
        in_specs=[
            pl.BlockSpec((d_in, blk), lambda i: (0, i)),
            pl.BlockSpec((1, blk), lambda i: (0, i)),
            pl.BlockSpec((ts_t.shape[0], blk), lambda i: (0, i)),
            pl.BlockSpec((num_modes, d_in), lambda i: (0, 0)),
            pl.BlockSpec((d_out, d_in), lambda i: (0, 0)),
            pl.BlockSpec((d_out, d_in), lambda i: (0, 0)),
            pl.BlockSpec((d_out, d_in), lambda i: (0, 0)),
            pl.BlockSpec((d_in, ts_t.shape[0]), lambda i: (0, 0)),
            pl.BlockSpec((d_in, 1), lambda i: (0, 0)),
            pl.BlockSpec((d_out, 1), lambda i: (0, 0)),
        ],
        out_specs=pl.BlockSpec((d_out, blk), lambda i: (0, i)),
        out_shape=jax.ShapeDtypeStruct((d_out, batch), jnp.float32),
    )(u_t, mode_t, ts_t, mode_table, w_user, w_mode, w_time,
      time_W, time_b_c, pref_b_c)


def kernel(user_id, transport_mode, timestamp, user_table, mode_table,
           time_W, time_b, pref_W, pref_b):
    batch = user_id.shape[0]
    d = user_table.shape[1]
    d_out = pref_W.shape[0]
    b_per_w = batch // NUM_WORKERS
    table128 = _tc_repack(user_table.T)
    uid3 = user_id.reshape(NUM_WORKERS, b_per_w // CHUNK, CHUNK)
    u_t = _sc_gather_compact(table128, uid3)
    out_t = _tc_merge_t(
        u_t,
        transport_mode.astype(jnp.float32).reshape(1, batch),
        timestamp.T,
        mode_table,
        pref_W[:, 0:d],
        pref_W[:, d:2 * d],
        pref_W[:, 2 * d:3 * d],
        time_W,
        time_b.reshape(d, 1),
        pref_b.reshape(d_out, 1),
    )
    return out_t.T


# speedup vs baseline: 4.6433x; 1.0172x over previous
import functools

import jax
import jax.numpy as jnp
from jax import lax
from jax.experimental import pallas as pl
from jax.experimental.pallas import tpu as pltpu
from jax.experimental.pallas import tpu_sc as plsc

NUM_CORES = 2
NUM_SUBCORES = 16
NUM_WORKERS = NUM_CORES * NUM_SUBCORES
CHUNK = 128
PACK = 4
W_BLK = 65536
C_BLK = W_BLK // PACK
L = 16


def _tc_repack(table_t):
    d, v = table_t.shape
    w = W_BLK
    grid = ((v + w - 1) // w,)
    c = w // PACK

    def body(t_ref, o_ref):
        xs = jnp.concatenate(
            [t_ref[:, pl.ds(p * c, c)] for p in range(PACK)], axis=0
        )
        rows = lax.broadcasted_iota(jnp.int32, (PACK * d, PACK * d), 0)
        cols = lax.broadcasted_iota(jnp.int32, (PACK * d, PACK * d), 1)
        e = (cols == PACK * (rows % d) + rows // d).astype(jnp.float32)
        o_ref[...] = lax.dot_general(xs, e, (((0,), (0,)), ((), ())),
                                     preferred_element_type=jnp.float32)

    return pl.pallas_call(
        body,
        grid=grid,
        in_specs=[pl.BlockSpec((d, w), lambda i: (0, i))],
        out_specs=pl.BlockSpec((w // PACK, PACK * d), lambda i: (i, 0)),
        out_shape=jax.ShapeDtypeStruct((grid[0] * (w // PACK), PACK * d),
                                       jnp.float32),
    )(table_t)


@jax.jit
def _sc_gather_compact(table128, uid3):
    n_chunks = uid3.shape[1]
    b_per_w = n_chunks * CHUNK
    batch = NUM_WORKERS * b_per_w
    d = 32
    mesh = plsc.VectorSubcoreMesh(core_axis_name="c", subcore_axis_name="s")

    @functools.partial(
        pl.kernel,
        mesh=mesh,
        compiler_params=pltpu.CompilerParams(needs_layout_passes=False),
        out_type=jax.ShapeDtypeStruct((d, batch), jnp.float32),
        scratch_types=[
            pltpu.VMEM((n_chunks, CHUNK), jnp.int32),
            pltpu.VMEM((n_chunks, CHUNK), jnp.int32),
            pltpu.VMEM((b_per_w, PACK * d), jnp.float32),
            pltpu.VMEM((d, b_per_w), jnp.float32),
            pltpu.SemaphoreType.DMA,
        ],
    )
    def k(table_hbm, uid_hbm, out_hbm, uid_v, rid_v, rows_v, outt_v, sem):
        wid = lax.axis_index("s") * NUM_CORES + lax.axis_index("c")
        base = wid * b_per_w
        pltpu.sync_copy(uid_hbm.at[wid], uid_v)
        for c in range(n_chunks):
            for i in range(CHUNK // L):
                u = uid_v[c, pl.ds(i * L, L)]
                rid_v[c, pl.ds(i * L, L)] = (
                    (u >> 16) * C_BLK + (u & (C_BLK - 1)))
        copies = []
        for c in range(n_chunks):
            copies.append(
                pltpu.async_copy(
                    table_hbm.at[rid_v.at[c]],
                    rows_v.at[pl.ds(c * CHUNK, CHUNK)],
                    sem,
                )
            )
        kiota = lax.broadcasted_iota(jnp.int32, (L,), 0)
        for c in range(n_chunks):
            copies[c].wait()
            for i in range(CHUNK // L):
                rowbase = c * CHUNK + i * L
                u = uid_v[c, pl.ds(i * L, L)]
                p = (u >> 14) & 3
                rows16 = kiota + rowbase
                for kk in range(d):
                    vals = plsc.load_gather(rows_v, [rows16, PACK * kk + p])
                    outt_v[kk, pl.ds(rowbase, L)] = vals
        pltpu.sync_copy(outt_v, out_hbm.at[:, pl.ds(base, b_per_w)])

    return k(table128, uid3)


def _tc_merge_t(u_t, mode_t, ts_t, mode_table, w_user, w_mode, w_time,
                time_W, time_b_c, pref_b_c):
    d_in, batch = u_t.shape
    d_out = w_user.shape[0]
    num_modes = mode_table.shape[0]
    blk = 4096
    grid = (batch // blk,)

    def body(u_ref, m_ref, t_ref, mt_ref, wu_ref, wm_ref, wt_ref,
             tw_ref, tb_ref, pb_ref, o_ref):
        user_c = lax.dot_general(wu_ref[...], u_ref[...],
                                 (((1,), (0,)), ((), ())),
                                 preferred_element_type=jnp.float32)
        oh = (lax.broadcasted_iota(jnp.int32, (num_modes, blk), 0)
              .astype(jnp.float32) == m_ref[...]).astype(jnp.float32)
        m2 = lax.dot_general(wm_ref[...], mt_ref[...],
                             (((1,), (1,)), ((), ())),
                             preferred_element_type=jnp.float32)
        mode_c = lax.dot_general(m2, oh, (((1,), (0,)), ((), ())),
                                 preferred_element_type=jnp.float32)
        wc = lax.dot_general(wt_ref[...], tw_ref[...],
                             (((1,), (0,)), ((), ())),
                             preferred_element_type=jnp.float32)
        time_c = lax.dot_general(wc, t_ref[...], (((1,), (0,)), ((), ())),
                                 preferred_element_type=jnp.float32)
        bias = lax.dot_general(wt_ref[...], tb_ref[...],
                               (((1,), (0,)), ((), ())),
                               preferred_element_type=jnp.float32) + pb_ref[...]
        o_ref[...] = user_c + mode_c + time_c + bias

    return pl.pallas_call(
        body,
        grid=grid,
        in_specs=[
            pl.BlockSpec((d_in, blk), lambda i: (0, i)),
            pl.BlockSpec((1, blk), lambda i: (0, i)),
            pl.BlockSpec((ts_t.shape[0], blk), lambda i: (0, i)),
            pl.BlockSpec((num_modes, d_in), lambda i: (0, 0)),
            pl.BlockSpec((d_out, d_in), lambda i: (0, 0)),
            pl.BlockSpec((d_out, d_in), lambda i: (0, 0)),
            pl.BlockSpec((d_out, d_in), lambda i: (0, 0)),
            pl.BlockSpec((d_in, ts_t.shape[0]), lambda i: (0, 0)),
            pl.BlockSpec((d_in, 1), lambda i: (0, 0)),
            pl.BlockSpec((d_out, 1), lambda i: (0, 0)),
        ],
        out_specs=pl.BlockSpec((d_out, blk), lambda i: (0, i)),
        out_shape=jax.ShapeDtypeStruct((d_out, batch), jnp.float32),
    )(u_t, mode_t, ts_t, mode_table, w_user, w_mode, w_time,
      time_W, time_b_c, pref_b_c)


def kernel(user_id, transport_mode, timestamp, user_table, mode_table,
           time_W, time_b, pref_W, pref_b):
    batch = user_id.shape[0]
    d = user_table.shape[1]
    d_out = pref_W.shape[0]
    b_per_w = batch // NUM_WORKERS
    table128 = _tc_repack(user_table.T)
    uid3 = user_id.reshape(NUM_WORKERS, b_per_w // CHUNK, CHUNK)
    u_t = _sc_gather_compact(table128, uid3)
    out_t = _tc_merge_t(
        u_t,
        transport_mode.astype(jnp.float32).reshape(1, batch),
        timestamp.T,
        mode_table,
        pref_W[:, 0:d],
        pref_W[:, d:2 * d],
        pref_W[:, 2 * d:3 * d],
        time_W,
        time_b.reshape(d, 1),
        pref_b.reshape(d_out, 1),
    )
    return out_t.T
